# baseline (speedup 1.0000x reference)
import functools

import jax
import jax.numpy as jnp
from jax import lax
from jax.experimental import pallas as pl
from jax.experimental.pallas import tpu as pltpu
from jax.experimental.pallas import tpu_sc as plsc

_PADW = 64
_OUTW = 64


def _make_sc_kernel(B, D, C, end_size):
    mesh = plsc.VectorSubcoreMesh(
        core_axis_name="c", subcore_axis_name="s", num_cores=2, num_subcores=16)
    n_workers = 32
    xpw = (B * D) // n_workers
    orpw = xpw * C
    group_rows = B * D * C
    L = 16

    @functools.partial(
        pl.kernel,
        out_type=jax.ShapeDtypeStruct((3 * group_rows, _OUTW), jnp.float32),
        mesh=mesh,
        compiler_params=pltpu.CompilerParams(
            use_tc_tiling_on_sc=False, needs_layout_passes=False),
        scratch_types=dict(
            snv=pltpu.VMEM((L,), jnp.int32),
            bA0=pltpu.VMEM((orpw, D), jnp.float32),
            bA1=pltpu.VMEM((orpw, D), jnp.float32),
            bB=pltpu.VMEM((orpw, D), jnp.float32),
            stg=pltpu.VMEM((2, D * C, L), jnp.float32),
            oA=pltpu.VMEM((orpw, _OUTW), jnp.float32),
            oB=pltpu.VMEM((orpw, _OUTW), jnp.float32),
            oC=pltpu.VMEM((orpw, _OUTW), jnp.float32),
            sem0=pltpu.SemaphoreType.DMA,
            sem1=pltpu.SemaphoreType.DMA,
            sem2=pltpu.SemaphoreType.DMA,
            sem3=pltpu.SemaphoreType.DMA,
        ),
    )
    def sc_kernel(vol, snum, out, snv, bA0, bA1, bB, stg,
                  oA, oB, oC, sem0, sem1, sem2, sem3):
        cid = lax.axis_index("c")
        sid = lax.axis_index("s")
        wid = sid * 2 + cid
        b = lax.shift_right_logical(wid, 4)
        x0 = (wid & 15) * xpw

        pltpu.sync_copy(snum, snv)
        snvec = snv[:]
        sn = snvec[0]
        s = lax.shift_right_logical(sn * D, 8)
        s_al = pl.multiple_of(
            lax.shift_left(lax.shift_right_logical(s, 3), 3), 8)
        ls = s - s_al

        rA = ((b * D + s) * D + x0) * C
        cA0 = pltpu.async_copy(vol.at[pl.ds(rA, orpw), pl.ds(0, D)], bA0, sem0)
        cA1 = pltpu.async_copy(vol.at[pl.ds(rA + D * C, orpw), pl.ds(0, D)],
                               bA1, sem0)
        cBs = []
        for xi in range(xpw):
            rB = ((b * D + x0 + xi) * D + s) * C
            cBs.append(pltpu.async_copy(
                vol.at[pl.ds(rB, C), pl.ds(0, D)],
                bB.at[pl.ds(xi * C, C)], sem1))

        def stage(xi):
            rX = (b * D + x0 + xi) * (D * C)
            return pltpu.async_copy(
                vol.at[pl.ds(rX, D * C), pl.ds(s_al, L)],
                stg.at[xi & 1], sem2)

        cS = stage(0)

        fvec = ((snvec * D) & (end_size - 1)).astype(jnp.float32) * (1.0 / end_size)
        gvec = fvec + 1.0
        iota = lax.iota(jnp.int32, L)
        lsv = jnp.full((L,), ls, jnp.int32)

        cA0.wait()
        cA1.wait()
        for cc in cBs:
            cc.wait()

        def ab_body(r, carry):
            for k in range(D // L):
                r0 = bA0[r, pl.ds(k * L, L)]
                r1 = bA1[r, pl.ds(k * L, L)]
                rb = bB[r, pl.ds(k * L, L)]
                oA[r, pl.ds(k * L, L)] = gvec * (r1 - r0)
                oB[r, pl.ds(k * L, L)] = gvec * (r1 - rb)
            return carry
        lax.fori_loop(0, orpw, ab_body, 0)

        for xi in range(xpw):
            cS.wait()
            if xi + 1 < xpw:
                cS = stage(xi + 1)

            def c_body(ch, carry, xi=xi):
                for q in range(D // L):
                    rows = (q * L + iota) * C + ch
                    c0 = plsc.load_gather(stg.at[xi & 1], [rows, lsv])
                    c1 = plsc.load_gather(stg.at[xi & 1], [rows, lsv + 1])
                    oC[xi * C + ch, pl.ds(q * L, L)] = fvec * (c1 - c0) + c0
                return carry
            lax.fori_loop(0, C, c_body, 0)

        obase = (b * D + x0) * C
        wA = pltpu.async_copy(oA, out.at[pl.ds(obase, orpw)], sem3)
        wB = pltpu.async_copy(oB, out.at[pl.ds(group_rows + obase, orpw)], sem3)
        wC = pltpu.async_copy(oC, out.at[pl.ds(2 * group_rows + obase, orpw)], sem3)
        wA.wait()
        wB.wait()
        wC.wait()

    return sc_kernel


def kernel(volume, slice_ax, slice_num, upsmp):
    B, D, _, _, C = volume.shape
    end_size = D * 4
    sc = _make_sc_kernel(B, D, C, end_size)
    vt = volume.transpose(0, 1, 2, 4, 3)
    v128 = vt.reshape(B * D * D * C, _PADW)
    sn16 = jnp.broadcast_to(slice_num.reshape(-1)[:1].astype(jnp.int32), (16,))
    out = sc(v128, sn16)
    return out.reshape(3 * B, D, C, D).transpose(0, 1, 3, 2)


# speedup vs baseline: 1.1328x; 1.1328x over previous
import functools

import jax
import jax.numpy as jnp
from jax import lax
from jax.experimental import pallas as pl
from jax.experimental.pallas import tpu as pltpu
from jax.experimental.pallas import tpu_sc as plsc

_PADW = 128
_OUTW = 64


def _make_sc_kernel(B, D, C, end_size):
    mesh = plsc.VectorSubcoreMesh(
        core_axis_name="c", subcore_axis_name="s", num_cores=2, num_subcores=16)
    n_workers = 32
    xpw = (B * D) // n_workers
    orpw = xpw * C
    group_rows = B * D * C
    L = 16

    @functools.partial(
        pl.kernel,
        out_type=jax.ShapeDtypeStruct((3 * group_rows, _OUTW), jnp.float32),
        mesh=mesh,
        compiler_params=pltpu.CompilerParams(
            use_tc_tiling_on_sc=False, needs_layout_passes=False),
        scratch_types=dict(
            snv=pltpu.VMEM((L,), jnp.int32),
            bA0=pltpu.VMEM((1, orpw, D), jnp.float32),
            bA1=pltpu.VMEM((1, orpw, D), jnp.float32),
            bB=pltpu.VMEM((xpw, C, D), jnp.float32),
            stg=pltpu.VMEM((2, 1, D * C, L), jnp.float32),
            oA=pltpu.VMEM((orpw, _OUTW), jnp.float32),
            oB=pltpu.VMEM((orpw, _OUTW), jnp.float32),
            oC=pltpu.VMEM((orpw, _OUTW), jnp.float32),
            sem0=pltpu.SemaphoreType.DMA,
            sem1=pltpu.SemaphoreType.DMA,
            sem2=pltpu.SemaphoreType.DMA,
            sem3=pltpu.SemaphoreType.DMA,
        ),
    )
    def sc_kernel(vol, snum, out, snv, bA0, bA1, bB, stg,
                  oA, oB, oC, sem0, sem1, sem2, sem3):
        cid = lax.axis_index("c")
        sid = lax.axis_index("s")
        wid = sid * 2 + cid
        b = lax.shift_right_logical(wid, 4)
        x0 = (wid & 15) * xpw

        pltpu.sync_copy(snum, snv)
        snvec = snv[:]
        sn = snvec[0]
        s = lax.shift_right_logical(sn * D, 8)
        s_al = pl.multiple_of(
            lax.shift_left(lax.shift_right_logical(s, 3), 3), 8)
        ls = s - s_al

        cA0 = pltpu.async_copy(
            vol.at[pl.ds(b * D + s, 1), pl.ds(x0 * C, orpw), pl.ds(0, D)],
            bA0, sem0)
        cA1 = pltpu.async_copy(
            vol.at[pl.ds(b * D + s + 1, 1), pl.ds(x0 * C, orpw), pl.ds(0, D)],
            bA1, sem0)
        cB = pltpu.async_copy(
            vol.at[pl.ds(b * D + x0, xpw), pl.ds(s * C, C), pl.ds(0, D)],
            bB, sem1)

        def stage(xi):
            return pltpu.async_copy(
                vol.at[pl.ds(b * D + x0 + xi, 1), :, pl.ds(s_al, L)],
                stg.at[xi & 1], sem2)

        cS = stage(0)

        fvec = ((snvec * D) & (end_size - 1)).astype(jnp.float32) * (1.0 / end_size)
        gvec = fvec + 1.0
        iota = lax.iota(jnp.int32, L)
        lsv = jnp.full((L,), ls, jnp.int32)

        cA0.wait()
        cA1.wait()
        cB.wait()

        def ab_body(r, carry):
            for k in range(D // L):
                r0 = bA0[0, r, pl.ds(k * L, L)]
                r1 = bA1[0, r, pl.ds(k * L, L)]
                rb = bB[lax.shift_right_logical(r, 4), r & (C - 1),
                        pl.ds(k * L, L)]
                oA[r, pl.ds(k * L, L)] = gvec * (r1 - r0)
                oB[r, pl.ds(k * L, L)] = gvec * (r1 - rb)
            return carry
        lax.fori_loop(0, orpw, ab_body, 0)

        obase = (b * D + x0) * C
        wA = pltpu.async_copy(oA, out.at[pl.ds(obase, orpw)], sem3)
        wB = pltpu.async_copy(oB, out.at[pl.ds(group_rows + obase, orpw)], sem3)

        for xi in range(xpw):
            cS.wait()
            if xi + 1 < xpw:
                cS = stage(xi + 1)

            def c_body(ch, carry, xi=xi):
                for q in range(D // L):
                    rows = (q * L + iota) * C + ch
                    c0 = plsc.load_gather(stg.at[xi & 1, 0], [rows, lsv])
                    c1 = plsc.load_gather(stg.at[xi & 1, 0], [rows, lsv + 1])
                    oC[xi * C + ch, pl.ds(q * L, L)] = fvec * (c1 - c0) + c0
                return carry
            lax.fori_loop(0, C, c_body, 0)

        wC = pltpu.async_copy(oC, out.at[pl.ds(2 * group_rows + obase, orpw)], sem3)
        wA.wait()
        wB.wait()
        wC.wait()

    return sc_kernel


def kernel(volume, slice_ax, slice_num, upsmp):
    B, D, _, _, C = volume.shape
    end_size = D * 4
    sc = _make_sc_kernel(B, D, C, end_size)
    vt = volume.transpose(0, 1, 2, 4, 3)
    vp = jnp.pad(vt, ((0, 0), (0, 0), (0, 0), (0, 0), (0, _PADW - D)))
    v128 = vp.reshape(B * D, D * C, _PADW)
    sn16 = jnp.broadcast_to(slice_num.reshape(-1)[:1].astype(jnp.int32), (16,))
    out = sc(v128, sn16)
    return out.reshape(3 * B, D, C, D).transpose(0, 1, 3, 2)


# speedup vs baseline: 1.1343x; 1.0013x over previous
import functools

import jax
import jax.numpy as jnp
from jax import lax
from jax.experimental import pallas as pl
from jax.experimental.pallas import tpu as pltpu
from jax.experimental.pallas import tpu_sc as plsc

_PADW = 128
_OUTW = 64


def _make_sc_kernel(B, D, C, end_size):
    mesh = plsc.VectorSubcoreMesh(
        core_axis_name="c", subcore_axis_name="s", num_cores=2, num_subcores=16)
    n_workers = 32
    xpw = (B * D) // n_workers
    orpw = xpw * C
    group_rows = B * D * C
    L = 16

    @functools.partial(
        pl.kernel,
        out_type=jax.ShapeDtypeStruct((3 * group_rows, _OUTW), jnp.float32),
        mesh=mesh,
        compiler_params=pltpu.CompilerParams(
            use_tc_tiling_on_sc=False, needs_layout_passes=False,
            skip_device_barrier=True),
        scratch_types=dict(
            snv=pltpu.VMEM((L,), jnp.int32),
            bA0=pltpu.VMEM((1, orpw, D), jnp.float32),
            bA1=pltpu.VMEM((1, orpw, D), jnp.float32),
            bB=pltpu.VMEM((xpw, C, D), jnp.float32),
            stg=pltpu.VMEM((2, 1, D * C, L), jnp.float32),
            oA=pltpu.VMEM((orpw, _OUTW), jnp.float32),
            oB=pltpu.VMEM((orpw, _OUTW), jnp.float32),
            oC=pltpu.VMEM((orpw, _OUTW), jnp.float32),
            sem0=pltpu.SemaphoreType.DMA,
            sem1=pltpu.SemaphoreType.DMA,
            sem2=pltpu.SemaphoreType.DMA,
            sem3=pltpu.SemaphoreType.DMA,
        ),
    )
    def sc_kernel(vol, snum, out, snv, bA0, bA1, bB, stg,
                  oA, oB, oC, sem0, sem1, sem2, sem3):
        cid = lax.axis_index("c")
        sid = lax.axis_index("s")
        wid = sid * 2 + cid
        b = lax.shift_right_logical(wid, 4)
        x0 = (wid & 15) * xpw

        pltpu.sync_copy(snum, snv)
        snvec = snv[:]
        sn = snvec[0]
        s = lax.shift_right_logical(sn * D, 8)
        s_al = pl.multiple_of(
            lax.shift_left(lax.shift_right_logical(s, 3), 3), 8)
        ls = s - s_al

        cA0 = pltpu.async_copy(
            vol.at[pl.ds(b * D + s, 1), pl.ds(x0 * C, orpw), pl.ds(0, D)],
            bA0, sem0)
        cA1 = pltpu.async_copy(
            vol.at[pl.ds(b * D + s + 1, 1), pl.ds(x0 * C, orpw), pl.ds(0, D)],
            bA1, sem0)
        cB = pltpu.async_copy(
            vol.at[pl.ds(b * D + x0, xpw), pl.ds(s * C, C), pl.ds(0, D)],
            bB, sem1)

        def stage(xi):
            return pltpu.async_copy(
                vol.at[pl.ds(b * D + x0 + xi, 1), :, pl.ds(s_al, L)],
                stg.at[xi & 1], sem2)

        cS = stage(0)

        fvec = ((snvec * D) & (end_size - 1)).astype(jnp.float32) * (1.0 / end_size)
        gvec = fvec + 1.0
        iota = lax.iota(jnp.int32, L)
        lsv = jnp.full((L,), ls, jnp.int32)

        cA0.wait()
        cA1.wait()
        cB.wait()

        def ab_body(r, carry):
            for k in range(D // L):
                r0 = bA0[0, r, pl.ds(k * L, L)]
                r1 = bA1[0, r, pl.ds(k * L, L)]
                rb = bB[lax.shift_right_logical(r, 4), r & (C - 1),
                        pl.ds(k * L, L)]
                oA[r, pl.ds(k * L, L)] = gvec * (r1 - r0)
                oB[r, pl.ds(k * L, L)] = gvec * (r1 - rb)
            return carry
        lax.fori_loop(0, orpw, ab_body, 0)

        obase = (b * D + x0) * C
        wA = pltpu.async_copy(oA, out.at[pl.ds(obase, orpw)], sem3)
        wB = pltpu.async_copy(oB, out.at[pl.ds(group_rows + obase, orpw)], sem3)

        for xi in range(xpw):
            cS.wait()
            if xi + 1 < xpw:
                cS = stage(xi + 1)

            def c_body(ch, carry, xi=xi):
                for q in range(D // L):
                    rows = (q * L + iota) * C + ch
                    c0 = plsc.load_gather(stg.at[xi & 1, 0], [rows, lsv])
                    c1 = plsc.load_gather(stg.at[xi & 1, 0], [rows, lsv + 1])
                    oC[xi * C + ch, pl.ds(q * L, L)] = fvec * (c1 - c0) + c0
                return carry
            lax.fori_loop(0, C, c_body, 0)

        wC = pltpu.async_copy(oC, out.at[pl.ds(2 * group_rows + obase, orpw)], sem3)
        wA.wait()
        wB.wait()
        wC.wait()

    return sc_kernel


def kernel(volume, slice_ax, slice_num, upsmp):
    B, D, _, _, C = volume.shape
    end_size = D * 4
    sc = _make_sc_kernel(B, D, C, end_size)
    vt = volume.transpose(0, 1, 2, 4, 3)
    vp = jnp.pad(vt, ((0, 0), (0, 0), (0, 0), (0, 0), (0, _PADW - D)))
    v128 = vp.reshape(B * D, D * C, _PADW)
    sn16 = jnp.broadcast_to(slice_num.reshape(-1)[:1].astype(jnp.int32), (16,))
    out = sc(v128, sn16)
    return out.reshape(3 * B, D, C, D).transpose(0, 1, 3, 2)


# speedup vs baseline: 1.1364x; 1.0019x over previous
import functools

import jax
import jax.numpy as jnp
from jax import lax
from jax.experimental import pallas as pl
from jax.experimental.pallas import tpu as pltpu
from jax.experimental.pallas import tpu_sc as plsc

_PADW = 128
_OUTW = 64


def _make_sc_kernel(B, D, C, end_size):
    mesh = plsc.VectorSubcoreMesh(
        core_axis_name="c", subcore_axis_name="s", num_cores=2, num_subcores=16)
    n_workers = 32
    xpw = (B * D) // n_workers
    orpw = xpw * C
    group_rows = B * D * C
    L = 16

    @functools.partial(
        pl.kernel,
        out_type=jax.ShapeDtypeStruct((3 * group_rows, _OUTW), jnp.float32),
        mesh=mesh,
        compiler_params=pltpu.CompilerParams(
            use_tc_tiling_on_sc=False, needs_layout_passes=False),
        scratch_types=dict(
            snv=pltpu.VMEM((L,), jnp.int32),
            bA0=pltpu.VMEM((1, orpw, D), jnp.float32),
            bA1=pltpu.VMEM((1, orpw, D), jnp.float32),
            bB=pltpu.VMEM((xpw, C, D), jnp.float32),
            stg=pltpu.VMEM((2, 1, D * C, L), jnp.float32),
            oA=pltpu.VMEM((orpw, _OUTW), jnp.float32),
            oB=pltpu.VMEM((orpw, _OUTW), jnp.float32),
            oC=pltpu.VMEM((orpw, _OUTW), jnp.float32),
            sem0=pltpu.SemaphoreType.DMA,
            sem1=pltpu.SemaphoreType.DMA,
            sem2=pltpu.SemaphoreType.DMA,
            sem3=pltpu.SemaphoreType.DMA,
        ),
    )
    def sc_kernel(vol, snum, out, snv, bA0, bA1, bB, stg,
                  oA, oB, oC, sem0, sem1, sem2, sem3):
        cid = lax.axis_index("c")
        sid = lax.axis_index("s")
        wid = sid * 2 + cid
        b = lax.shift_right_logical(wid, 4)
        x0 = (wid & 15) * xpw

        pltpu.sync_copy(snum, snv)
        snvec = snv[:]
        sn = snvec[0]
        s = lax.shift_right_logical(sn * D, 8)
        s_al = pl.multiple_of(
            lax.shift_left(lax.shift_right_logical(s, 3), 3), 8)
        ls = s - s_al

        cA0 = pltpu.async_copy(
            vol.at[pl.ds(b * D + s, 1), pl.ds(x0 * C, orpw), pl.ds(0, D)],
            bA0, sem0)
        cA1 = pltpu.async_copy(
            vol.at[pl.ds(b * D + s + 1, 1), pl.ds(x0 * C, orpw), pl.ds(0, D)],
            bA1, sem0)
        cB = pltpu.async_copy(
            vol.at[pl.ds(b * D + x0, xpw), pl.ds(s * C, C), pl.ds(0, D)],
            bB, sem1)

        def stage(xi):
            return pltpu.async_copy(
                vol.at[pl.ds(b * D + x0 + xi, 1), :, pl.ds(s_al, L)],
                stg.at[xi & 1], sem2)

        cS = stage(0)

        fvec = ((snvec * D) & (end_size - 1)).astype(jnp.float32) * (1.0 / end_size)
        gvec = fvec + 1.0
        iota = lax.iota(jnp.int32, L)
        lsv = jnp.full((L,), ls, jnp.int32)

        cA0.wait()
        cA1.wait()
        cB.wait()

        def ab_body(r, carry):
            for k in range(D // L):
                r0 = bA0[0, r, pl.ds(k * L, L)]
                r1 = bA1[0, r, pl.ds(k * L, L)]
                rb = bB[lax.shift_right_logical(r, 4), r & (C - 1),
                        pl.ds(k * L, L)]
                oA[r, pl.ds(k * L, L)] = gvec * (r1 - r0)
                oB[r, pl.ds(k * L, L)] = gvec * (r1 - rb)
            return carry
        lax.fori_loop(0, orpw, ab_body, 0)

        obase = (b * D + x0) * C
        wA = pltpu.async_copy(oA, out.at[pl.ds(obase, orpw)], sem3)
        wB = pltpu.async_copy(oB, out.at[pl.ds(group_rows + obase, orpw)], sem3)

        for xi in range(xpw):
            cS.wait()
            if xi + 1 < xpw:
                cS = stage(xi + 1)

            def c_body(ch, carry, xi=xi):
                for q in range(D // L):
                    rows = (q * L + iota) * C + ch
                    c0 = plsc.load_gather(stg.at[xi & 1, 0], [rows, lsv])
                    c1 = plsc.load_gather(stg.at[xi & 1, 0], [rows, lsv + 1])
                    oC[xi * C + ch, pl.ds(q * L, L)] = fvec * (c1 - c0) + c0
                return carry
            lax.fori_loop(0, C, c_body, 0)

        wC = pltpu.async_copy(oC, out.at[pl.ds(2 * group_rows + obase, orpw)], sem3)
        wA.wait()
        wB.wait()
        wC.wait()

    return sc_kernel


def kernel(volume, slice_ax, slice_num, upsmp):
    B, D, _, _, C = volume.shape
    end_size = D * 4
    sc = _make_sc_kernel(B, D, C, end_size)
    vt = volume.transpose(0, 1, 2, 4, 3)
    vp = jnp.pad(vt, ((0, 0), (0, 0), (0, 0), (0, 0), (0, _PADW - D)))
    v128 = vp.reshape(B * D, D * C, _PADW)
    sn16 = jnp.broadcast_to(slice_num.reshape(-1)[:1].astype(jnp.int32), (16,))
    out = sc(v128, sn16)
    return out.reshape(3 * B, D, C, D).transpose(0, 1, 3, 2)
